# Initial kernel scaffold; baseline (speedup 1.0000x reference)
import jax
import jax.numpy as jnp
from jax.experimental import pallas as pl


def kernel(feat, edge_index):
    raise NotImplementedError("write your pallas kernel here")


# speedup vs baseline: 3.7875x; 3.7875x over previous
import functools

import jax
import jax.numpy as jnp
from jax import lax
from jax.experimental import pallas as pl
from jax.experimental.pallas import tpu as pltpu
from jax.experimental.pallas import tpu_sc as plsc

NC = 2
NS = 16
NW = NC * NS


def _pick_chunk(epw):
    for c in range(128, 7, -8):
        if epw % c == 0:
            return c
    raise ValueError(f"no valid chunk for {epw} edges per worker")


@functools.cache
def _degrees_kernel(n, e):
    assert e % NW == 0
    epw = e // NW
    chunk = _pick_chunk(epw)
    nch = epw // chunk
    zrows = ((n + NS - 1) // NS + 15) // 16 * 16
    zlast = n - zrows * (NS - 1)
    assert zlast > 0 and zrows % 16 == 0
    mesh = plsc.VectorSubcoreMesh(core_axis_name="c", subcore_axis_name="s")

    def body(src_hbm, dst_hbm, out_hbm, idx_v, ones_v, zeros_v, cnt_src, cnt_dst):
        cid = lax.axis_index("c")
        sid = lax.axis_index("s")
        wid = sid * NC + cid
        base = wid * epw

        def fill_z(i, carry):
            zeros_v[pl.ds(i * 16, 16)] = jnp.zeros((16,), jnp.float32)
            return carry

        lax.fori_loop(0, zrows // 16, fill_z, 0)

        def fill_o(i, carry):
            ones_v[pl.ds(i * 16, 16)] = jnp.ones((16,), jnp.float32)
            return carry

        lax.fori_loop(0, chunk // 16, fill_o, 0)

        @pl.when(sid < NS - 1)
        def _zero_main():
            pltpu.sync_copy(zeros_v, cnt_src.at[pl.ds(sid * zrows, zrows)])
            pltpu.sync_copy(zeros_v, cnt_dst.at[pl.ds(sid * zrows, zrows)])

        @pl.when(sid == NS - 1)
        def _zero_tail():
            pltpu.sync_copy(zeros_v.at[pl.ds(0, zlast)],
                            cnt_src.at[pl.ds((NS - 1) * zrows, zlast)])
            pltpu.sync_copy(zeros_v.at[pl.ds(0, zlast)],
                            cnt_dst.at[pl.ds((NS - 1) * zrows, zlast)])

        plsc.subcore_barrier()

        def step(j, carry):
            off = base + j * chunk
            pltpu.sync_copy(src_hbm.at[pl.ds(off, chunk)], idx_v.at[0])
            pltpu.sync_copy(dst_hbm.at[pl.ds(off, chunk)], idx_v.at[1])
            pltpu.sync_copy(ones_v, cnt_src.at[idx_v.at[0]], add=True)
            pltpu.sync_copy(ones_v, cnt_dst.at[idx_v.at[1]], add=True)
            return carry

        lax.fori_loop(0, nch, step, 0)

        plsc.subcore_barrier()

        @pl.when(sid == 0)
        def _writeout():
            pltpu.sync_copy(cnt_src, out_hbm.at[cid, 0])
            pltpu.sync_copy(cnt_dst, out_hbm.at[cid, 1])

    return pl.kernel(
        body,
        out_type=jax.ShapeDtypeStruct((NC, 2, n), jnp.float32),
        mesh=mesh,
        scratch_types=[
            pltpu.VMEM((2, chunk), jnp.int32),
            pltpu.VMEM((chunk,), jnp.float32),
            pltpu.VMEM((zrows,), jnp.float32),
            pltpu.VMEM_SHARED((n,), jnp.float32),
            pltpu.VMEM_SHARED((n,), jnp.float32),
        ],
    )


@functools.cache
def _agg_kernel(n, e, d):
    assert e % NW == 0 and d % 16 == 0
    epw = e // NW
    chunk = _pick_chunk(epw)
    nch = epw // chunk
    rmain = ((n + NS - 1) // NS + 7) // 8 * 8
    rtail = n - rmain * (NS - 1)
    assert 0 < rtail <= rmain and rtail % 8 == 0
    mesh = plsc.VectorSubcoreMesh(core_axis_name="c", subcore_axis_name="s")

    def zero_rows(rows_v, agg_sh, row0, nrows):
        t = 0
        while (t + 1) * chunk <= nrows:
            pltpu.sync_copy(rows_v, agg_sh.at[pl.ds(row0 + t * chunk, chunk)])
            t += 1
        rem = nrows - t * chunk
        if rem:
            pltpu.sync_copy(rows_v.at[pl.ds(0, rem)],
                            agg_sh.at[pl.ds(row0 + t * chunk, rem)])

    def body(sfeat_hbm, src_hbm, dst_hbm, out_hbm, idx_v, rows_v, agg_sh, sem):
        cid = lax.axis_index("c")
        sid = lax.axis_index("s")
        wid = sid * NC + cid
        base = wid * epw

        def fill_z(i, carry):
            r = i // (d // 16)
            c = i % (d // 16)
            rows_v[r, pl.ds(c * 16, 16)] = jnp.zeros((16,), jnp.float32)
            return carry

        lax.fori_loop(0, chunk * (d // 16), fill_z, 0)

        @pl.when(sid < NS - 1)
        def _zero_main():
            zero_rows(rows_v, agg_sh, sid * rmain, rmain)

        @pl.when(sid == NS - 1)
        def _zero_tail():
            zero_rows(rows_v, agg_sh, (NS - 1) * rmain, rtail)

        plsc.subcore_barrier()

        def step(j, carry):
            off = base + j * chunk
            pltpu.sync_copy(src_hbm.at[pl.ds(off, chunk)], idx_v.at[0])
            pltpu.sync_copy(dst_hbm.at[pl.ds(off, chunk)], idx_v.at[1])
            pltpu.async_copy(sfeat_hbm.at[idx_v.at[0]], rows_v, sem).wait()
            pltpu.sync_copy(rows_v, agg_sh.at[idx_v.at[1]], add=True)
            return carry

        lax.fori_loop(0, nch, step, 0)

        plsc.subcore_barrier()

        @pl.when(sid < NS - 1)
        def _write_main():
            pltpu.sync_copy(agg_sh.at[pl.ds(sid * rmain, rmain)],
                            out_hbm.at[cid, pl.ds(sid * rmain, rmain)])

        @pl.when(sid == NS - 1)
        def _write_tail():
            pltpu.sync_copy(agg_sh.at[pl.ds((NS - 1) * rmain, rtail)],
                            out_hbm.at[cid, pl.ds((NS - 1) * rmain, rtail)])

    return pl.kernel(
        body,
        out_type=jax.ShapeDtypeStruct((NC, n, d), jnp.float32),
        mesh=mesh,
        scratch_types=[
            pltpu.VMEM((2, chunk), jnp.int32),
            pltpu.VMEM((chunk, d), jnp.float32),
            pltpu.VMEM_SHARED((n, d), jnp.float32),
            pltpu.SemaphoreType.DMA,
        ],
    )


def _scale_body(feat_ref, cnt_ref, out_ref):
    deg = cnt_ref[0, 0] + cnt_ref[1, 0]
    norm = lax.rsqrt(jnp.maximum(deg, 1.0))
    out_ref[...] = feat_ref[...] * norm


def _score_body(feat_ref, agg_ref, cnt_ref, out_ref):
    indeg = cnt_ref[0, 1] + cnt_ref[1, 1]
    dn = lax.rsqrt(jnp.maximum(indeg, 1.0))
    agg = agg_ref[0] + agg_ref[1]
    x = feat_ref[...] - agg * dn
    out_ref[...] = jnp.sum(jnp.abs(x), axis=1, keepdims=True)


@functools.cache
def _scale_kernel(n, d):
    return pl.pallas_call(
        _scale_body,
        out_shape=jax.ShapeDtypeStruct((n, d), jnp.float32),
    )


@functools.cache
def _score_kernel(n, d):
    return pl.pallas_call(
        _score_body,
        out_shape=jax.ShapeDtypeStruct((n, 1), jnp.float32),
    )


def kernel(feat, edge_index):
    n, d = feat.shape
    e = edge_index.shape[1]
    src = edge_index[0]
    dst = edge_index[1]
    counts = _degrees_kernel(n, e)(src, dst)
    counts4 = counts.reshape(NC, 2, n, 1)
    sfeat = _scale_kernel(n, d)(feat, counts4)
    agg2 = _agg_kernel(n, e, d)(sfeat, src, dst)
    score = _score_kernel(n, d)(feat, agg2, counts4)
    return score.reshape(n)
